# baseline (device time: 98829 ns/iter reference)
import jax
import jax.numpy as jnp
from jax import lax
from jax.experimental import pallas as pl
from jax.experimental.pallas import tpu as pltpu


def kernel(x, assign, W1, W2):
    t, d = x.shape
    e_loc, _, f = W1.shape

    xb = x.astype(jnp.bfloat16)
    w1b = W1.astype(jnp.bfloat16)
    w2b = W2.astype(jnp.bfloat16)
    a2 = assign.reshape(t, 1)

    def body(x_ref, a_ref, w1_ref, w2_ref, out_ref,
             xpeer, apeer, rsend, rrecv, send_sems, recv_sems):
        my_x = lax.axis_index("x")
        my_y = lax.axis_index("y")
        peer = (my_x, 1 - my_y)

        barrier = pltpu.get_barrier_semaphore()
        pl.semaphore_signal(barrier, inc=1, device_id=peer,
                            device_id_type=pl.DeviceIdType.MESH)
        pl.semaphore_wait(barrier, 1)

        rdma_x = pltpu.make_async_remote_copy(
            src_ref=x_ref, dst_ref=xpeer,
            send_sem=send_sems.at[0], recv_sem=recv_sems.at[0],
            device_id=peer, device_id_type=pl.DeviceIdType.MESH)
        rdma_x.start()
        rdma_a = pltpu.make_async_remote_copy(
            src_ref=a_ref, dst_ref=apeer,
            send_sem=send_sems.at[1], recv_sem=recv_sems.at[1],
            device_id=peer, device_id_type=pl.DeviceIdType.MESH)
        rdma_a.start()

        def block_contrib(xblk, ablk):
            acc = jnp.zeros((t, d), jnp.float32)
            for e in range(e_loc):
                ge = my_y * e_loc + e
                xm = jnp.where(ablk == ge, xblk, jnp.array(0, xblk.dtype))
                h = jnp.dot(xm, w1_ref[e], preferred_element_type=jnp.float32)
                h = jnp.maximum(h, 0.0).astype(jnp.bfloat16)
                acc = acc + jnp.dot(h, w2_ref[e],
                                    preferred_element_type=jnp.float32)
            return acc

        out_ref[...] = block_contrib(x_ref[...], a_ref[...])

        rdma_x.wait()
        rdma_a.wait()

        rsend[...] = block_contrib(xpeer[...], apeer[...]).astype(jnp.bfloat16)
        rdma_r = pltpu.make_async_remote_copy(
            src_ref=rsend, dst_ref=rrecv,
            send_sem=send_sems.at[2], recv_sem=recv_sems.at[2],
            device_id=peer, device_id_type=pl.DeviceIdType.MESH)
        rdma_r.start()
        rdma_r.wait()

        out_ref[...] = out_ref[...] + rrecv[...].astype(jnp.float32)

    return pl.pallas_call(
        body,
        out_shape=jax.ShapeDtypeStruct((t, d), jnp.float32),
        in_specs=[
            pl.BlockSpec(memory_space=pltpu.VMEM),
            pl.BlockSpec(memory_space=pltpu.VMEM),
            pl.BlockSpec(memory_space=pltpu.VMEM),
            pl.BlockSpec(memory_space=pltpu.VMEM),
        ],
        out_specs=pl.BlockSpec(memory_space=pltpu.VMEM),
        scratch_shapes=[
            pltpu.VMEM((t, d), jnp.bfloat16),
            pltpu.VMEM((t, 1), jnp.int32),
            pltpu.VMEM((t, d), jnp.bfloat16),
            pltpu.VMEM((t, d), jnp.bfloat16),
            pltpu.SemaphoreType.DMA((3,)),
            pltpu.SemaphoreType.DMA((3,)),
        ],
        compiler_params=pltpu.CompilerParams(collective_id=0),
    )(xb, a2, w1b, w2b)


# device time: 77955 ns/iter; 1.2678x vs baseline; 1.2678x over previous
import jax
import jax.numpy as jnp
from jax import lax
from jax.experimental import pallas as pl
from jax.experimental.pallas import tpu as pltpu


def kernel(x, assign, W1, W2):
    t, d = x.shape
    e_loc, _, f = W1.shape
    th = t // 2

    xb = x.astype(jnp.bfloat16)
    w1b = W1.astype(jnp.bfloat16)
    w2b = W2.astype(jnp.bfloat16)
    a2 = assign.reshape(t, 1)

    def body(x_ref, a_ref, w1_ref, w2_ref, out_ref,
             xsend, asend, xpeer, apeer, rsend, rret, osend, orecv,
             send_sems, recv_sems):
        my_x = lax.axis_index("x")
        my_y = lax.axis_index("y")
        ypeer_id = (my_x, 1 - my_y)
        xpeer_id = (1 - my_x, my_y)
        row0 = pl.multiple_of(my_x * th, th)

        barrier = pltpu.get_barrier_semaphore()
        for nbr in (ypeer_id, xpeer_id):
            pl.semaphore_signal(barrier, inc=1, device_id=nbr,
                                device_id_type=pl.DeviceIdType.MESH)
        pl.semaphore_wait(barrier, 2)

        xsend[...] = x_ref[pl.ds(row0, th), :]
        asend[...] = a_ref[pl.ds(row0, th), :]
        rdma_x = pltpu.make_async_remote_copy(
            src_ref=xsend, dst_ref=xpeer,
            send_sem=send_sems.at[0], recv_sem=recv_sems.at[0],
            device_id=ypeer_id, device_id_type=pl.DeviceIdType.MESH)
        rdma_x.start()
        rdma_a = pltpu.make_async_remote_copy(
            src_ref=asend, dst_ref=apeer,
            send_sem=send_sems.at[1], recv_sem=recv_sems.at[1],
            device_id=ypeer_id, device_id_type=pl.DeviceIdType.MESH)
        rdma_a.start()

        def block_contrib(xblk, ablk):
            acc = jnp.zeros((th, d), jnp.float32)
            for e in range(e_loc):
                ge = my_y * e_loc + e
                xm = jnp.where(ablk == ge, xblk, jnp.array(0, xblk.dtype))
                h = jnp.dot(xm, w1_ref[e], preferred_element_type=jnp.float32)
                h = jnp.maximum(h, 0.0).astype(jnp.bfloat16)
                acc = acc + jnp.dot(h, w2_ref[e],
                                    preferred_element_type=jnp.float32)
            return acc

        lacc = block_contrib(xsend[...], asend[...])

        rdma_x.wait()
        rdma_a.wait()

        rsend[...] = block_contrib(xpeer[...], apeer[...]).astype(jnp.bfloat16)
        rdma_r = pltpu.make_async_remote_copy(
            src_ref=rsend, dst_ref=rret,
            send_sem=send_sems.at[2], recv_sem=recv_sems.at[2],
            device_id=ypeer_id, device_id_type=pl.DeviceIdType.MESH)
        rdma_r.start()
        rdma_r.wait()

        myhalf = lacc + rret[...].astype(jnp.float32)
        out_ref[pl.ds(row0, th), :] = myhalf
        osend[...] = myhalf.astype(jnp.bfloat16)
        rdma_o = pltpu.make_async_remote_copy(
            src_ref=osend, dst_ref=orecv,
            send_sem=send_sems.at[3], recv_sem=recv_sems.at[3],
            device_id=xpeer_id, device_id_type=pl.DeviceIdType.MESH)
        rdma_o.start()
        rdma_o.wait()
        other0 = pl.multiple_of((1 - my_x) * th, th)
        out_ref[pl.ds(other0, th), :] = orecv[...].astype(jnp.float32)

    return pl.pallas_call(
        body,
        out_shape=jax.ShapeDtypeStruct((t, d), jnp.float32),
        in_specs=[
            pl.BlockSpec(memory_space=pltpu.VMEM),
            pl.BlockSpec(memory_space=pltpu.VMEM),
            pl.BlockSpec(memory_space=pltpu.VMEM),
            pl.BlockSpec(memory_space=pltpu.VMEM),
        ],
        out_specs=pl.BlockSpec(memory_space=pltpu.VMEM),
        scratch_shapes=[
            pltpu.VMEM((th, d), jnp.bfloat16),
            pltpu.VMEM((th, 1), jnp.int32),
            pltpu.VMEM((th, d), jnp.bfloat16),
            pltpu.VMEM((th, 1), jnp.int32),
            pltpu.VMEM((th, d), jnp.bfloat16),
            pltpu.VMEM((th, d), jnp.bfloat16),
            pltpu.VMEM((th, d), jnp.bfloat16),
            pltpu.VMEM((th, d), jnp.bfloat16),
            pltpu.SemaphoreType.DMA((4,)),
            pltpu.SemaphoreType.DMA((4,)),
        ],
        compiler_params=pltpu.CompilerParams(collective_id=0),
    )(xb, a2, w1b, w2b)


# device time: 66173 ns/iter; 1.4935x vs baseline; 1.1780x over previous
import jax
import jax.numpy as jnp
from jax import lax
from jax.experimental import pallas as pl
from jax.experimental.pallas import tpu as pltpu


def kernel(x, assign, W1, W2):
    t, d = x.shape
    e_loc, _, f = W1.shape
    th = t // 2

    a2 = assign.reshape(t, 1)

    def body(x_ref, a_ref, w1_hbm, w2_hbm, out_ref,
             s1, s2, w1b, w2b,
             xsend, asend, xpeer, apeer, rsend, rret, osend, orecv,
             send_sems, recv_sems, local_sems):
        my_x = lax.axis_index("x")
        my_y = lax.axis_index("y")
        ypeer_id = (my_x, 1 - my_y)
        xpeer_id = (1 - my_x, my_y)
        row0 = pl.multiple_of(my_x * th, th)

        barrier = pltpu.get_barrier_semaphore()
        for nbr in (ypeer_id, xpeer_id):
            pl.semaphore_signal(barrier, inc=1, device_id=nbr,
                                device_id_type=pl.DeviceIdType.MESH)
        pl.semaphore_wait(barrier, 2)

        xsend[...] = x_ref[pl.ds(row0, th), :].astype(jnp.bfloat16)
        asend[...] = a_ref[pl.ds(row0, th), :]
        rdma_x = pltpu.make_async_remote_copy(
            src_ref=xsend, dst_ref=xpeer,
            send_sem=send_sems.at[0], recv_sem=recv_sems.at[0],
            device_id=ypeer_id, device_id_type=pl.DeviceIdType.MESH)
        rdma_x.start()
        rdma_a = pltpu.make_async_remote_copy(
            src_ref=asend, dst_ref=apeer,
            send_sem=send_sems.at[1], recv_sem=recv_sems.at[1],
            device_id=ypeer_id, device_id_type=pl.DeviceIdType.MESH)
        rdma_a.start()

        c10 = pltpu.make_async_copy(w1_hbm.at[0], s1, local_sems.at[0])
        c20 = pltpu.make_async_copy(w2_hbm.at[0], s2, local_sems.at[1])
        c10.start()
        c20.start()
        c10.wait()
        w1b[0] = s1[...].astype(jnp.bfloat16)
        c20.wait()
        w2b[0] = s2[...].astype(jnp.bfloat16)
        c11 = pltpu.make_async_copy(w1_hbm.at[1], s1, local_sems.at[0])
        c21 = pltpu.make_async_copy(w2_hbm.at[1], s2, local_sems.at[1])
        c11.start()
        c21.start()

        def expert_contrib(xblk, ablk, e):
            ge = my_y * e_loc + e
            xm = jnp.where(ablk == ge, xblk, jnp.array(0, xblk.dtype))
            h = jnp.dot(xm, w1b[e], preferred_element_type=jnp.float32)
            h = jnp.maximum(h, 0.0).astype(jnp.bfloat16)
            return jnp.dot(h, w2b[e], preferred_element_type=jnp.float32)

        rdma_x.wait()
        rdma_a.wait()
        ret0 = expert_contrib(xpeer[...], apeer[...], 0)
        c11.wait()
        w1b[1] = s1[...].astype(jnp.bfloat16)
        c21.wait()
        w2b[1] = s2[...].astype(jnp.bfloat16)
        ret1 = expert_contrib(xpeer[...], apeer[...], 1)
        rsend[...] = (ret0 + ret1).astype(jnp.bfloat16)
        rdma_r = pltpu.make_async_remote_copy(
            src_ref=rsend, dst_ref=rret,
            send_sem=send_sems.at[2], recv_sem=recv_sems.at[2],
            device_id=ypeer_id, device_id_type=pl.DeviceIdType.MESH)
        rdma_r.start()

        lacc = (expert_contrib(xsend[...], asend[...], 0)
                + expert_contrib(xsend[...], asend[...], 1))

        rdma_r.wait()

        myhalf = lacc + rret[...].astype(jnp.float32)
        out_ref[pl.ds(row0, th), :] = myhalf
        osend[...] = myhalf.astype(jnp.bfloat16)
        rdma_o = pltpu.make_async_remote_copy(
            src_ref=osend, dst_ref=orecv,
            send_sem=send_sems.at[3], recv_sem=recv_sems.at[3],
            device_id=xpeer_id, device_id_type=pl.DeviceIdType.MESH)
        rdma_o.start()
        rdma_o.wait()
        other0 = pl.multiple_of((1 - my_x) * th, th)
        out_ref[pl.ds(other0, th), :] = orecv[...].astype(jnp.float32)

    return pl.pallas_call(
        body,
        out_shape=jax.ShapeDtypeStruct((t, d), jnp.float32),
        in_specs=[
            pl.BlockSpec(memory_space=pltpu.VMEM),
            pl.BlockSpec(memory_space=pltpu.VMEM),
            pl.BlockSpec(memory_space=pl.ANY),
            pl.BlockSpec(memory_space=pl.ANY),
        ],
        out_specs=pl.BlockSpec(memory_space=pltpu.VMEM),
        scratch_shapes=[
            pltpu.VMEM((d, f), jnp.float32),
            pltpu.VMEM((f, d), jnp.float32),
            pltpu.VMEM((e_loc, d, f), jnp.bfloat16),
            pltpu.VMEM((e_loc, f, d), jnp.bfloat16),
            pltpu.VMEM((th, d), jnp.bfloat16),
            pltpu.VMEM((th, 1), jnp.int32),
            pltpu.VMEM((th, d), jnp.bfloat16),
            pltpu.VMEM((th, 1), jnp.int32),
            pltpu.VMEM((th, d), jnp.bfloat16),
            pltpu.VMEM((th, d), jnp.bfloat16),
            pltpu.VMEM((th, d), jnp.bfloat16),
            pltpu.VMEM((th, d), jnp.bfloat16),
            pltpu.SemaphoreType.DMA((4,)),
            pltpu.SemaphoreType.DMA((4,)),
            pltpu.SemaphoreType.DMA((2,)),
        ],
        compiler_params=pltpu.CompilerParams(
            collective_id=0, vmem_limit_bytes=60 * 1024 * 1024),
    )(x, a2, W1, W2)


# device time: 52266 ns/iter; 1.8909x vs baseline; 1.2661x over previous
import jax
import jax.numpy as jnp
from jax import lax
from jax.experimental import pallas as pl
from jax.experimental.pallas import tpu as pltpu

NT = 2


def kernel(x, assign, W1, W2):
    t, d = x.shape
    e_loc, _, f = W1.shape
    th = t // 2
    tt = th // NT

    a2 = assign.reshape(t, 1)

    def body(x_ref, a_ref, w1_hbm, w2_hbm, out_ref,
             s1, s2, w1b, w2b,
             xsend, asend, xpeer, apeer, rsend, rret, osend, orecv,
             sems_xs, sems_xr, sems_as, sems_ar,
             sems_rs, sems_rr, sems_os, sems_or, local_sems):
        my_x = lax.axis_index("x")
        my_y = lax.axis_index("y")
        ypeer_id = (my_x, 1 - my_y)
        xpeer_id = (1 - my_x, my_y)
        row0 = pl.multiple_of(my_x * th, th)
        other0 = pl.multiple_of((1 - my_x) * th, th)

        barrier = pltpu.get_barrier_semaphore()
        for nbr in (ypeer_id, xpeer_id):
            pl.semaphore_signal(barrier, inc=1, device_id=nbr,
                                device_id_type=pl.DeviceIdType.MESH)
        pl.semaphore_wait(barrier, 2)

        asend[...] = a_ref[pl.ds(row0, th), :]
        rdma_a = pltpu.make_async_remote_copy(
            src_ref=asend, dst_ref=apeer,
            send_sem=sems_as.at[0], recv_sem=sems_ar.at[0],
            device_id=ypeer_id, device_id_type=pl.DeviceIdType.MESH)
        rdma_a.start()

        rdma_x = []
        for k in range(NT):
            sl = pl.ds(k * tt, tt)
            xsend[sl, :] = x_ref[pl.ds(row0 + k * tt, tt), :].astype(
                jnp.bfloat16)
            r = pltpu.make_async_remote_copy(
                src_ref=xsend.at[sl], dst_ref=xpeer.at[sl],
                send_sem=sems_xs.at[k], recv_sem=sems_xr.at[k],
                device_id=ypeer_id, device_id_type=pl.DeviceIdType.MESH)
            r.start()
            rdma_x.append(r)

        stages = (s1, s2)

        def w_chunk(c):
            e, half, is_w2 = c // 4, (c // 2) % 2, c % 2
            dsl = pl.ds(half * d, d)
            if is_w2:
                return (w2_hbm.at[e, dsl, :], lambda v: w2b.__setitem__(
                    (e, dsl, slice(None)), v))
            return (w1_hbm.at[e, :, dsl], lambda v: w1b.__setitem__(
                (e, slice(None), dsl), v))

        def w_start(c):
            src, _ = w_chunk(c)
            cp = pltpu.make_async_copy(src, stages[c % 2],
                                       local_sems.at[c % 2])
            cp.start()
            return cp

        pending = [w_start(0), w_start(1)]
        for c in range(8):
            pending[c % 2].wait()
            _, store = w_chunk(c)
            store(stages[c % 2][...].astype(jnp.bfloat16))
            if c + 2 < 8:
                pending[c % 2] = w_start(c + 2)

        def contrib(xblk, ablk):
            acc = None
            for e in range(e_loc):
                ge = my_y * e_loc + e
                xm = jnp.where(ablk == ge, xblk, jnp.array(0, xblk.dtype))
                h = jnp.dot(xm, w1b[e], preferred_element_type=jnp.float32)
                h = jnp.maximum(h, 0.0).astype(jnp.bfloat16)
                o = jnp.dot(h, w2b[e], preferred_element_type=jnp.float32)
                acc = o if acc is None else acc + o
            return acc

        rdma_a.wait()
        rdma_r = []
        for k in range(NT):
            sl = pl.ds(k * tt, tt)
            rdma_x[k].wait()
            rsend[sl, :] = contrib(xpeer[sl, :], apeer[sl, :]).astype(
                jnp.bfloat16)
            r = pltpu.make_async_remote_copy(
                src_ref=rsend.at[sl], dst_ref=rret.at[sl],
                send_sem=sems_rs.at[k], recv_sem=sems_rr.at[k],
                device_id=ypeer_id, device_id_type=pl.DeviceIdType.MESH)
            r.start()
            rdma_r.append(r)

        rdma_o = []
        for k in range(NT):
            sl = pl.ds(k * tt, tt)
            l = contrib(xsend[sl, :], asend[sl, :])
            rdma_r[k].wait()
            mh = l + rret[sl, :].astype(jnp.float32)
            out_ref[pl.ds(row0 + k * tt, tt), :] = mh
            osend[sl, :] = mh.astype(jnp.bfloat16)
            r = pltpu.make_async_remote_copy(
                src_ref=osend.at[sl], dst_ref=orecv.at[sl],
                send_sem=sems_os.at[k], recv_sem=sems_or.at[k],
                device_id=xpeer_id, device_id_type=pl.DeviceIdType.MESH)
            r.start()
            rdma_o.append(r)

        for k in range(NT):
            rdma_o[k].wait()
            sl = pl.ds(k * tt, tt)
            out_ref[pl.ds(other0 + k * tt, tt), :] = orecv[sl, :].astype(
                jnp.float32)

    return pl.pallas_call(
        body,
        out_shape=jax.ShapeDtypeStruct((t, d), jnp.float32),
        in_specs=[
            pl.BlockSpec(memory_space=pltpu.VMEM),
            pl.BlockSpec(memory_space=pltpu.VMEM),
            pl.BlockSpec(memory_space=pl.ANY),
            pl.BlockSpec(memory_space=pl.ANY),
        ],
        out_specs=pl.BlockSpec(memory_space=pltpu.VMEM),
        scratch_shapes=[
            pltpu.VMEM((d, d), jnp.float32),
            pltpu.VMEM((d, d), jnp.float32),
            pltpu.VMEM((e_loc, d, f), jnp.bfloat16),
            pltpu.VMEM((e_loc, f, d), jnp.bfloat16),
            pltpu.VMEM((th, d), jnp.bfloat16),
            pltpu.VMEM((th, 1), jnp.int32),
            pltpu.VMEM((th, d), jnp.bfloat16),
            pltpu.VMEM((th, 1), jnp.int32),
            pltpu.VMEM((th, d), jnp.bfloat16),
            pltpu.VMEM((th, d), jnp.bfloat16),
            pltpu.VMEM((th, d), jnp.bfloat16),
            pltpu.VMEM((th, d), jnp.bfloat16),
            pltpu.SemaphoreType.DMA((NT,)),
            pltpu.SemaphoreType.DMA((NT,)),
            pltpu.SemaphoreType.DMA((1,)),
            pltpu.SemaphoreType.DMA((1,)),
            pltpu.SemaphoreType.DMA((NT,)),
            pltpu.SemaphoreType.DMA((NT,)),
            pltpu.SemaphoreType.DMA((NT,)),
            pltpu.SemaphoreType.DMA((NT,)),
            pltpu.SemaphoreType.DMA((2,)),
        ],
        compiler_params=pltpu.CompilerParams(
            collective_id=0, vmem_limit_bytes=60 * 1024 * 1024),
    )(x, a2, W1, W2)


# device time: 51801 ns/iter; 1.9079x vs baseline; 1.0090x over previous
import jax
import jax.numpy as jnp
from jax import lax
from jax.experimental import pallas as pl
from jax.experimental.pallas import tpu as pltpu

NT = 4


def kernel(x, assign, W1, W2):
    t, d = x.shape
    e_loc, _, f = W1.shape
    th = t // 2
    tt = th // NT

    a2 = assign.reshape(t, 1)

    def body(x_ref, a_ref, w1_hbm, w2_hbm, out_ref,
             s1, s2, w1b, w2b,
             xsend, asend, xpeer, apeer, rsend, rret, osend, orecv,
             sems_xs, sems_xr, sems_as, sems_ar,
             sems_rs, sems_rr, sems_os, sems_or, local_sems):
        my_x = lax.axis_index("x")
        my_y = lax.axis_index("y")
        ypeer_id = (my_x, 1 - my_y)
        xpeer_id = (1 - my_x, my_y)
        row0 = pl.multiple_of(my_x * th, th)
        other0 = pl.multiple_of((1 - my_x) * th, th)

        barrier = pltpu.get_barrier_semaphore()
        for nbr in (ypeer_id, xpeer_id):
            pl.semaphore_signal(barrier, inc=1, device_id=nbr,
                                device_id_type=pl.DeviceIdType.MESH)
        pl.semaphore_wait(barrier, 2)

        asend[...] = a_ref[pl.ds(row0, th), :]
        rdma_a = pltpu.make_async_remote_copy(
            src_ref=asend, dst_ref=apeer,
            send_sem=sems_as.at[0], recv_sem=sems_ar.at[0],
            device_id=ypeer_id, device_id_type=pl.DeviceIdType.MESH)
        rdma_a.start()

        rdma_x = []
        for k in range(NT):
            sl = pl.ds(k * tt, tt)
            xsend[sl, :] = x_ref[pl.ds(row0 + k * tt, tt), :].astype(
                jnp.bfloat16)
            r = pltpu.make_async_remote_copy(
                src_ref=xsend.at[sl], dst_ref=xpeer.at[sl],
                send_sem=sems_xs.at[k], recv_sem=sems_xr.at[k],
                device_id=ypeer_id, device_id_type=pl.DeviceIdType.MESH)
            r.start()
            rdma_x.append(r)

        stages = (s1, s2)

        def w_chunk(c):
            e, half, is_w2 = c // 4, (c // 2) % 2, c % 2
            dsl = pl.ds(half * d, d)
            if is_w2:
                return (w2_hbm.at[e, dsl, :], lambda v: w2b.__setitem__(
                    (e, dsl, slice(None)), v))
            return (w1_hbm.at[e, :, dsl], lambda v: w1b.__setitem__(
                (e, slice(None), dsl), v))

        def w_start(c):
            src, _ = w_chunk(c)
            cp = pltpu.make_async_copy(src, stages[c % 2],
                                       local_sems.at[c % 2])
            cp.start()
            return cp

        pending = [w_start(0), w_start(1)]
        for c in range(8):
            pending[c % 2].wait()
            _, store = w_chunk(c)
            store(stages[c % 2][...].astype(jnp.bfloat16))
            if c + 2 < 8:
                pending[c % 2] = w_start(c + 2)

        def contrib(xblk, ablk):
            acc = None
            for e in range(e_loc):
                ge = my_y * e_loc + e
                xm = jnp.where(ablk == ge, xblk, jnp.array(0, xblk.dtype))
                h = jnp.dot(xm, w1b[e], preferred_element_type=jnp.float32)
                h = jnp.maximum(h, 0.0).astype(jnp.bfloat16)
                o = jnp.dot(h, w2b[e], preferred_element_type=jnp.float32)
                acc = o if acc is None else acc + o
            return acc

        rdma_a.wait()
        rdma_r = []
        for k in range(NT):
            sl = pl.ds(k * tt, tt)
            rdma_x[k].wait()
            rsend[sl, :] = contrib(xpeer[sl, :], apeer[sl, :]).astype(
                jnp.bfloat16)
            r = pltpu.make_async_remote_copy(
                src_ref=rsend.at[sl], dst_ref=rret.at[sl],
                send_sem=sems_rs.at[k], recv_sem=sems_rr.at[k],
                device_id=ypeer_id, device_id_type=pl.DeviceIdType.MESH)
            r.start()
            rdma_r.append(r)

        rdma_o = []
        for k in range(NT):
            sl = pl.ds(k * tt, tt)
            l = contrib(xsend[sl, :], asend[sl, :])
            rdma_r[k].wait()
            mh = l + rret[sl, :].astype(jnp.float32)
            out_ref[pl.ds(row0 + k * tt, tt), :] = mh
            osend[sl, :] = mh.astype(jnp.bfloat16)
            r = pltpu.make_async_remote_copy(
                src_ref=osend.at[sl], dst_ref=orecv.at[sl],
                send_sem=sems_os.at[k], recv_sem=sems_or.at[k],
                device_id=xpeer_id, device_id_type=pl.DeviceIdType.MESH)
            r.start()
            rdma_o.append(r)

        for k in range(NT):
            rdma_o[k].wait()
            sl = pl.ds(k * tt, tt)
            out_ref[pl.ds(other0 + k * tt, tt), :] = orecv[sl, :].astype(
                jnp.float32)

    return pl.pallas_call(
        body,
        out_shape=jax.ShapeDtypeStruct((t, d), jnp.float32),
        in_specs=[
            pl.BlockSpec(memory_space=pltpu.VMEM),
            pl.BlockSpec(memory_space=pltpu.VMEM),
            pl.BlockSpec(memory_space=pl.ANY),
            pl.BlockSpec(memory_space=pl.ANY),
        ],
        out_specs=pl.BlockSpec(memory_space=pltpu.VMEM),
        scratch_shapes=[
            pltpu.VMEM((d, d), jnp.float32),
            pltpu.VMEM((d, d), jnp.float32),
            pltpu.VMEM((e_loc, d, f), jnp.bfloat16),
            pltpu.VMEM((e_loc, f, d), jnp.bfloat16),
            pltpu.VMEM((th, d), jnp.bfloat16),
            pltpu.VMEM((th, 1), jnp.int32),
            pltpu.VMEM((th, d), jnp.bfloat16),
            pltpu.VMEM((th, 1), jnp.int32),
            pltpu.VMEM((th, d), jnp.bfloat16),
            pltpu.VMEM((th, d), jnp.bfloat16),
            pltpu.VMEM((th, d), jnp.bfloat16),
            pltpu.VMEM((th, d), jnp.bfloat16),
            pltpu.SemaphoreType.DMA((NT,)),
            pltpu.SemaphoreType.DMA((NT,)),
            pltpu.SemaphoreType.DMA((1,)),
            pltpu.SemaphoreType.DMA((1,)),
            pltpu.SemaphoreType.DMA((NT,)),
            pltpu.SemaphoreType.DMA((NT,)),
            pltpu.SemaphoreType.DMA((NT,)),
            pltpu.SemaphoreType.DMA((NT,)),
            pltpu.SemaphoreType.DMA((2,)),
        ],
        compiler_params=pltpu.CompilerParams(
            collective_id=0, vmem_limit_bytes=60 * 1024 * 1024),
    )(x, a2, W1, W2)


# device time: 49512 ns/iter; 1.9961x vs baseline; 1.0462x over previous
import jax
import jax.numpy as jnp
from jax import lax
from jax.experimental import pallas as pl
from jax.experimental.pallas import tpu as pltpu

NT = 4


def kernel(x, assign, W1, W2):
    t, d = x.shape
    e_loc, _, f = W1.shape
    th = t // 2
    tt = th // NT

    my_y_out = lax.axis_index("y")
    ge_loc = my_y_out * e_loc + jnp.arange(e_loc)
    ge_rem = (1 - my_y_out) * e_loc + jnp.arange(e_loc)
    oh_loc = (assign[:, None] == ge_loc[None, :]).astype(jnp.bfloat16)
    oh_rem = (assign[:, None] == ge_rem[None, :]).astype(jnp.bfloat16)

    def body(x_ref, oh_ref, ohr_ref, w1_hbm, w2_hbm, out_ref,
             s1, s2, w1b, w2b,
             xsend, ohsend, xpeer, ohpeer, rsend, rret,
             sems_xs, sems_xr, sems_as, sems_ar,
             sems_rs, sems_rr, sems_os, sems_or, local_sems):
        my_x = lax.axis_index("x")
        my_y = lax.axis_index("y")
        ypeer_id = (my_x, 1 - my_y)
        xpeer_id = (1 - my_x, my_y)
        row0 = pl.multiple_of(my_x * th, th)

        barrier = pltpu.get_barrier_semaphore()
        for nbr in (ypeer_id, xpeer_id):
            pl.semaphore_signal(barrier, inc=1, device_id=nbr,
                                device_id_type=pl.DeviceIdType.MESH)
        pl.semaphore_wait(barrier, 2)

        ohsend[...] = ohr_ref[pl.ds(row0, th), :]
        rdma_a = pltpu.make_async_remote_copy(
            src_ref=ohsend, dst_ref=ohpeer,
            send_sem=sems_as.at[0], recv_sem=sems_ar.at[0],
            device_id=ypeer_id, device_id_type=pl.DeviceIdType.MESH)
        rdma_a.start()

        rdma_x = []
        for k in range(NT):
            sl = pl.ds(k * tt, tt)
            xsend[sl, :] = x_ref[pl.ds(row0 + k * tt, tt), :].astype(
                jnp.bfloat16)
            r = pltpu.make_async_remote_copy(
                src_ref=xsend.at[sl], dst_ref=xpeer.at[sl],
                send_sem=sems_xs.at[k], recv_sem=sems_xr.at[k],
                device_id=ypeer_id, device_id_type=pl.DeviceIdType.MESH)
            r.start()
            rdma_x.append(r)

        stages = (s1, s2)

        def w_chunk(c):
            e, half, is_w2 = c // 4, (c // 2) % 2, c % 2
            dsl = pl.ds(half * d, d)
            if is_w2:
                return (w2_hbm.at[e, dsl, :], lambda v: w2b.__setitem__(
                    (e, dsl, slice(None)), v))
            return (w1_hbm.at[e, :, dsl], lambda v: w1b.__setitem__(
                (e, slice(None), dsl), v))

        def w_start(c):
            src, _ = w_chunk(c)
            cp = pltpu.make_async_copy(src, stages[c % 2],
                                       local_sems.at[c % 2])
            cp.start()
            return cp

        pending = [w_start(0), w_start(1)]
        for c in range(8):
            pending[c % 2].wait()
            _, store = w_chunk(c)
            store(stages[c % 2][...].astype(jnp.bfloat16))
            if c + 2 < 8:
                pending[c % 2] = w_start(c + 2)

        def contrib(xblk, ohblk):
            acc = None
            for e in range(e_loc):
                xm = xblk * ohblk[:, e:e + 1]
                h = jnp.dot(xm, w1b[e], preferred_element_type=jnp.float32)
                h = jnp.maximum(h, 0.0).astype(jnp.bfloat16)
                o = jnp.dot(h, w2b[e], preferred_element_type=jnp.float32)
                acc = o if acc is None else acc + o
            return acc

        rdma_a.wait()
        rdma_r = []
        for k in range(NT):
            sl = pl.ds(k * tt, tt)
            rdma_x[k].wait()
            rsend[sl, :] = contrib(xpeer[sl, :], ohpeer[sl, :]).astype(
                jnp.bfloat16)
            r = pltpu.make_async_remote_copy(
                src_ref=rsend.at[sl], dst_ref=rret.at[sl],
                send_sem=sems_rs.at[k], recv_sem=sems_rr.at[k],
                device_id=ypeer_id, device_id_type=pl.DeviceIdType.MESH)
            r.start()
            rdma_r.append(r)

        rdma_o = []
        for k in range(NT):
            sl = pl.ds(k * tt, tt)
            osl = pl.ds(row0 + k * tt, tt)
            l = contrib(xsend[sl, :], oh_ref[pl.ds(row0 + k * tt, tt), :])
            rdma_r[k].wait()
            out_ref[osl, :] = (l + rret[sl, :].astype(jnp.float32)).astype(
                jnp.bfloat16)
            r = pltpu.make_async_remote_copy(
                src_ref=out_ref.at[osl], dst_ref=out_ref.at[osl],
                send_sem=sems_os.at[k], recv_sem=sems_or.at[k],
                device_id=xpeer_id, device_id_type=pl.DeviceIdType.MESH)
            r.start()
            rdma_o.append(r)

        for k in range(NT):
            rdma_o[k].wait()

    return pl.pallas_call(
        body,
        out_shape=jax.ShapeDtypeStruct((t, d), jnp.bfloat16),
        in_specs=[
            pl.BlockSpec(memory_space=pltpu.VMEM),
            pl.BlockSpec(memory_space=pltpu.VMEM),
            pl.BlockSpec(memory_space=pltpu.VMEM),
            pl.BlockSpec(memory_space=pl.ANY),
            pl.BlockSpec(memory_space=pl.ANY),
        ],
        out_specs=pl.BlockSpec(memory_space=pltpu.VMEM),
        scratch_shapes=[
            pltpu.VMEM((d, d), jnp.float32),
            pltpu.VMEM((d, d), jnp.float32),
            pltpu.VMEM((e_loc, d, f), jnp.bfloat16),
            pltpu.VMEM((e_loc, f, d), jnp.bfloat16),
            pltpu.VMEM((th, d), jnp.bfloat16),
            pltpu.VMEM((th, e_loc), jnp.bfloat16),
            pltpu.VMEM((th, d), jnp.bfloat16),
            pltpu.VMEM((th, e_loc), jnp.bfloat16),
            pltpu.VMEM((th, d), jnp.bfloat16),
            pltpu.VMEM((th, d), jnp.bfloat16),
            pltpu.SemaphoreType.DMA((NT,)),
            pltpu.SemaphoreType.DMA((NT,)),
            pltpu.SemaphoreType.DMA((1,)),
            pltpu.SemaphoreType.DMA((1,)),
            pltpu.SemaphoreType.DMA((NT,)),
            pltpu.SemaphoreType.DMA((NT,)),
            pltpu.SemaphoreType.DMA((NT,)),
            pltpu.SemaphoreType.DMA((NT,)),
            pltpu.SemaphoreType.DMA((2,)),
        ],
        compiler_params=pltpu.CompilerParams(
            collective_id=0, vmem_limit_bytes=60 * 1024 * 1024),
    )(x, oh_loc, oh_rem, W1, W2)


# device time: 48276 ns/iter; 2.0472x vs baseline; 1.0256x over previous
import jax
import jax.numpy as jnp
from jax import lax
from jax.experimental import pallas as pl
from jax.experimental.pallas import tpu as pltpu

NT = 4


def kernel(x, assign, W1, W2):
    t, d = x.shape
    e_loc, _, f = W1.shape
    th = t // 2
    tt = th // NT

    my_y_out = lax.axis_index("y")
    ge_all = jnp.concatenate([
        my_y_out * e_loc + jnp.arange(e_loc),
        (1 - my_y_out) * e_loc + jnp.arange(e_loc),
    ])
    oh_all = (assign[:, None] == ge_all[None, :]).astype(jnp.bfloat16)

    def body(x_ref, oh_ref, w1_hbm, w2_hbm, out_ref,
             s1, s2, w1b, w2b,
             xsend, ohsend, xpeer, ohpeer, rsend, rret, osend,
             sems_xs, sems_xr, sems_as, sems_ar,
             sems_rs, sems_rr, sems_os, sems_or, sems_st, local_sems):
        my_x = lax.axis_index("x")
        my_y = lax.axis_index("y")
        ypeer_id = (my_x, 1 - my_y)
        xpeer_id = (1 - my_x, my_y)
        row0 = pl.multiple_of(my_x * th, th)

        barrier = pltpu.get_barrier_semaphore()
        for nbr in (ypeer_id, xpeer_id):
            pl.semaphore_signal(barrier, inc=1, device_id=nbr,
                                device_id_type=pl.DeviceIdType.MESH)
        pl.semaphore_wait(barrier, 2)

        stages = (s1, s2)

        def w_chunk(c):
            e, half, is_w2 = c // 4, (c // 2) % 2, c % 2
            dsl = pl.ds(half * d, d)
            if is_w2:
                return (w2_hbm.at[e, dsl, :], lambda v: w2b.__setitem__(
                    (e, dsl, slice(None)), v))
            return (w1_hbm.at[e, :, dsl], lambda v: w1b.__setitem__(
                (e, slice(None), dsl), v))

        def w_start(c):
            src, _ = w_chunk(c)
            cp = pltpu.make_async_copy(src, stages[c % 2],
                                       local_sems.at[c % 2])
            cp.start()
            return cp

        pending = [w_start(0), w_start(1)]

        ohsend[...] = oh_ref[pl.ds(row0, th), 2:4]
        rdma_a = pltpu.make_async_remote_copy(
            src_ref=ohsend, dst_ref=ohpeer,
            send_sem=sems_as.at[0], recv_sem=sems_ar.at[0],
            device_id=ypeer_id, device_id_type=pl.DeviceIdType.MESH)
        rdma_a.start()

        rdma_x = []
        for k in range(NT):
            sl = pl.ds(k * tt, tt)
            xsend[sl, :] = x_ref[pl.ds(row0 + k * tt, tt), :].astype(
                jnp.bfloat16)
            r = pltpu.make_async_remote_copy(
                src_ref=xsend.at[sl], dst_ref=xpeer.at[sl],
                send_sem=sems_xs.at[k], recv_sem=sems_xr.at[k],
                device_id=ypeer_id, device_id_type=pl.DeviceIdType.MESH)
            r.start()
            rdma_x.append(r)

        for c in range(8):
            pending[c % 2].wait()
            _, store = w_chunk(c)
            store(stages[c % 2][...].astype(jnp.bfloat16))
            if c + 2 < 8:
                pending[c % 2] = w_start(c + 2)

        def contrib(xblk, ohblk):
            acc = None
            for e in range(e_loc):
                xm = xblk * ohblk[:, e:e + 1]
                h = jnp.dot(xm, w1b[e], preferred_element_type=jnp.float32)
                h = jnp.maximum(h, 0.0).astype(jnp.bfloat16)
                o = jnp.dot(h, w2b[e], preferred_element_type=jnp.float32)
                acc = o if acc is None else acc + o
            return acc

        rdma_a.wait()
        rdma_r = []
        for k in range(NT):
            sl = pl.ds(k * tt, tt)
            rdma_x[k].wait()
            rsend[sl, :] = contrib(xpeer[sl, :], ohpeer[sl, :]).astype(
                jnp.bfloat16)
            r = pltpu.make_async_remote_copy(
                src_ref=rsend.at[sl], dst_ref=rret.at[sl],
                send_sem=sems_rs.at[k], recv_sem=sems_rr.at[k],
                device_id=ypeer_id, device_id_type=pl.DeviceIdType.MESH)
            r.start()
            rdma_r.append(r)

        rdma_o, stores = [], []
        for k in range(NT):
            sl = pl.ds(k * tt, tt)
            osl = pl.ds(row0 + k * tt, tt)
            l = contrib(xsend[sl, :], oh_ref[pl.ds(row0 + k * tt, tt), 0:2])
            rdma_r[k].wait()
            osend[sl, :] = (l + rret[sl, :].astype(jnp.float32)).astype(
                jnp.bfloat16)
            st = pltpu.make_async_copy(osend.at[sl], out_ref.at[osl],
                                       sems_st.at[k])
            st.start()
            stores.append(st)
            r = pltpu.make_async_remote_copy(
                src_ref=osend.at[sl], dst_ref=out_ref.at[osl],
                send_sem=sems_os.at[k], recv_sem=sems_or.at[k],
                device_id=xpeer_id, device_id_type=pl.DeviceIdType.MESH)
            r.start()
            rdma_o.append(r)

        for k in range(NT):
            stores[k].wait()
            rdma_o[k].wait()

    return pl.pallas_call(
        body,
        out_shape=jax.ShapeDtypeStruct((t, d), jnp.bfloat16),
        in_specs=[
            pl.BlockSpec(memory_space=pltpu.VMEM),
            pl.BlockSpec(memory_space=pltpu.VMEM),
            pl.BlockSpec(memory_space=pl.ANY),
            pl.BlockSpec(memory_space=pl.ANY),
        ],
        out_specs=pl.BlockSpec(memory_space=pl.ANY),
        scratch_shapes=[
            pltpu.VMEM((d, d), jnp.float32),
            pltpu.VMEM((d, d), jnp.float32),
            pltpu.VMEM((e_loc, d, f), jnp.bfloat16),
            pltpu.VMEM((e_loc, f, d), jnp.bfloat16),
            pltpu.VMEM((th, d), jnp.bfloat16),
            pltpu.VMEM((th, e_loc), jnp.bfloat16),
            pltpu.VMEM((th, d), jnp.bfloat16),
            pltpu.VMEM((th, e_loc), jnp.bfloat16),
            pltpu.VMEM((th, d), jnp.bfloat16),
            pltpu.VMEM((th, d), jnp.bfloat16),
            pltpu.VMEM((th, d), jnp.bfloat16),
            pltpu.SemaphoreType.DMA((NT,)),
            pltpu.SemaphoreType.DMA((NT,)),
            pltpu.SemaphoreType.DMA((1,)),
            pltpu.SemaphoreType.DMA((1,)),
            pltpu.SemaphoreType.DMA((NT,)),
            pltpu.SemaphoreType.DMA((NT,)),
            pltpu.SemaphoreType.DMA((NT,)),
            pltpu.SemaphoreType.DMA((NT,)),
            pltpu.SemaphoreType.DMA((NT,)),
            pltpu.SemaphoreType.DMA((2,)),
        ],
        compiler_params=pltpu.CompilerParams(
            collective_id=0, vmem_limit_bytes=60 * 1024 * 1024),
    )(x, oh_all, W1, W2)
